# baseline (speedup 1.0000x reference)
import functools

import jax
import jax.numpy as jnp
from jax import lax
from jax.experimental import pallas as pl
from jax.experimental.pallas import tpu as pltpu
from jax.experimental.pallas import tpu_sc as plsc

VOCAB = 1000
EMBED_DIM = 64
HIDDEN_DIM = 32
LPAD = 56


def _table_body(emb_ref, w1_ref, b1_ref, w2_ref, b2_ref, out_ref):
    h = lax.dot_general(
        emb_ref[...], w1_ref[...], (((1,), (0,)), ((), ())),
        preferred_element_type=jnp.float32)
    h = jnp.maximum(h + b1_ref[...], 0.0)
    out_ref[...] = lax.dot_general(
        h, w2_ref[...], (((1,), (0,)), ((), ())),
        preferred_element_type=jnp.float32) + b2_ref[...]


def _compute_table(emb, W1, b1, W2, b2):
    return pl.pallas_call(
        _table_body,
        out_shape=jax.ShapeDtypeStruct((VOCAB, VOCAB), jnp.float32),
    )(emb, W1, b1.reshape(1, HIDDEN_DIM), W2, b2.reshape(1, VOCAB))


@functools.cache
def _make_gather(B, L, V):
    info = plsc.get_sparse_core_info()
    NC, NS = info.num_cores, info.num_subcores
    NW = NC * NS
    rows_per_w = B // NW
    assert B % NW == 0
    n_stagers = 8
    v_per_s = V // n_stagers
    mesh = plsc.VectorSubcoreMesh(core_axis_name="c", subcore_axis_name="s")

    @functools.partial(
        pl.kernel, mesh=mesh,
        compiler_params=pltpu.CompilerParams(use_tc_tiling_on_sc=False),
        out_type=jax.ShapeDtypeStruct((B, L, V), jnp.float32),
        scratch_types=[
            pltpu.VMEM_SHARED((V, V), jnp.float32),
            pltpu.VMEM((rows_per_w * LPAD,), jnp.int32),
            pltpu.VMEM((L, V), jnp.float32),
            pltpu.SemaphoreType.DMA,
        ],
    )
    def gather(table_hbm, idx_hbm, out_hbm, table_sh, idx_v, rows_v, gsem):
        sid = lax.axis_index("s")
        wid = sid * NC + lax.axis_index("c")
        base = wid * rows_per_w
        @pl.when(sid < n_stagers)
        def _():
            pltpu.sync_copy(table_hbm.at[pl.ds(sid * v_per_s, v_per_s)],
                            table_sh.at[pl.ds(sid * v_per_s, v_per_s)])
        pltpu.sync_copy(idx_hbm.at[pl.ds(base * LPAD, rows_per_w * LPAD)],
                        idx_v)
        plsc.subcore_barrier()

        for r in range(rows_per_w):
            pltpu.async_copy(
                table_sh.at[idx_v.at[pl.ds(r * LPAD, L)]], rows_v, gsem,
            ).wait()
            pltpu.sync_copy(rows_v, out_hbm.at[base + r])

    return gather


def kernel(x, emb, W1, b1, W2, b2):
    Bt, L = x.shape
    table = _compute_table(emb, W1, b1, W2, b2)
    xp = jnp.pad(x.astype(jnp.int32), ((0, 0), (0, LPAD - L))).reshape(-1)
    return _make_gather(Bt, L, VOCAB)(table, xp)


# speedup vs baseline: 1.4111x; 1.4111x over previous
import functools

import jax
import jax.numpy as jnp
from jax import lax
from jax.experimental import pallas as pl
from jax.experimental.pallas import tpu as pltpu
from jax.experimental.pallas import tpu_sc as plsc

VOCAB = 1000
EMBED_DIM = 64
HIDDEN_DIM = 32
DPAD = 1024
LPAD = 56
NPIECE = DPAD // 128
TAIL = VOCAB - 7 * 128


def _table_body(emb_ref, w1_ref, b1_ref, w2_ref, b2_ref, out_ref):
    h = lax.dot_general(
        emb_ref[...], w1_ref[...], (((1,), (0,)), ((), ())),
        preferred_element_type=jnp.float32)
    h = jnp.maximum(h + b1_ref[...], 0.0)
    out_ref[...] = lax.dot_general(
        h, w2_ref[...], (((1,), (0,)), ((), ())),
        preferred_element_type=jnp.float32) + b2_ref[...]


def _compute_table(emb, W1, b1, W2, b2):
    w2p = jnp.pad(W2, ((0, 0), (0, DPAD - VOCAB)))
    b2p = jnp.pad(b2, (0, DPAD - VOCAB)).reshape(1, DPAD)
    return pl.pallas_call(
        _table_body,
        out_shape=jax.ShapeDtypeStruct((VOCAB, DPAD), jnp.float32),
    )(emb, W1, b1.reshape(1, HIDDEN_DIM), w2p, b2p)


@functools.cache
def _make_gather(B, L, V):
    info = plsc.get_sparse_core_info()
    NC, NS = info.num_cores, info.num_subcores
    NW = NC * NS
    rows_per_w = B // NW
    assert B % NW == 0
    mesh = plsc.VectorSubcoreMesh(core_axis_name="c", subcore_axis_name="s")

    @functools.partial(
        pl.kernel, mesh=mesh,
        compiler_params=pltpu.CompilerParams(
            use_tc_tiling_on_sc=True, needs_layout_passes=False),
        out_type=jax.ShapeDtypeStruct((B, L, V), jnp.float32),
        scratch_types=[
            pltpu.VMEM((rows_per_w * NPIECE * LPAD,), jnp.int32),
            pltpu.VMEM((L, V), jnp.float32),
            pltpu.VMEM((L, 128), jnp.float32),
            pltpu.SemaphoreType.DMA,
            pltpu.SemaphoreType.DMA,
        ],
    )
    def gather(table_hbm, idx_hbm, out_hbm, idx_v, rows_v, tail_v,
               gsem, ssem):
        sid = lax.axis_index("s")
        wid = sid * NC + lax.axis_index("c")
        base = wid * rows_per_w
        n_idx = rows_per_w * NPIECE * LPAD
        pltpu.sync_copy(idx_hbm.at[pl.ds(wid * n_idx, n_idx)], idx_v)

        lane = lax.iota(jnp.int32, 16)
        lane8 = lane & 7
        low8 = lane < 8

        def merge_row(r, carry):
            for c in range(6):
                rows_v[r, pl.ds(7 * 128 + c * 16, 16)] = \
                    tail_v[r, pl.ds(c * 16, 16)]
            rvec = jnp.full((16,), r, dtype=jnp.int32)
            vals = plsc.load_gather(tail_v, [rvec, 96 + lane8], mask=low8)
            plsc.store_scatter(rows_v, [rvec, 992 + lane8], vals, mask=low8)
            return carry

        def row_body(r, carry):
            roff = pl.multiple_of(r * (NPIECE * LPAD), NPIECE * LPAD)
            hs = []
            for k in range(NPIECE - 1):
                hs.append(pltpu.async_copy(
                    table_hbm.at[idx_v.at[pl.ds(roff + k * LPAD, L)]],
                    rows_v.at[:, pl.ds(k * 128, 128)], gsem))
            hs.append(pltpu.async_copy(
                table_hbm.at[idx_v.at[pl.ds(roff + (NPIECE - 1) * LPAD, L)]],
                tail_v, gsem))
            for h in hs:
                h.wait()
            lax.fori_loop(0, L, merge_row, 0)
            pltpu.sync_copy(rows_v, out_hbm.at[base + r])
            return carry

        lax.fori_loop(0, rows_per_w, row_body, 0)

    return gather


def kernel(x, emb, W1, b1, W2, b2):
    Bt, L = x.shape
    tp = _compute_table(emb, W1, b1, W2, b2)
    table = tp.reshape(VOCAB * NPIECE, 128)
    xi = x.astype(jnp.int32)
    idx8 = (NPIECE * xi[:, None, :]
            + jnp.arange(NPIECE, dtype=jnp.int32)[None, :, None])
    idx8 = jnp.pad(idx8, ((0, 0), (0, 0), (0, LPAD - L))).reshape(-1)
    return _make_gather(Bt, L, VOCAB)(table, idx8)


# speedup vs baseline: 7.5911x; 5.3797x over previous
import functools

import jax
import jax.numpy as jnp
from jax import lax
from jax.experimental import pallas as pl

VOCAB = 1000
EMBED_DIM = 64
HIDDEN_DIM = 32


def _mlp_t_body(xt_ref, embt_ref, w1t_ref, b1_ref, w2t_ref, b2_ref, out_ref):
    xv = xt_ref[0]
    iota_v = lax.broadcasted_iota(jnp.int32, (VOCAB, xv.shape[1]), 0)
    m = jnp.where(iota_v == xv, 1.0, 0.0)
    et = lax.dot_general(
        embt_ref[...], m, (((1,), (0,)), ((), ())),
        preferred_element_type=jnp.float32)
    ht = lax.dot_general(
        w1t_ref[...], et, (((1,), (0,)), ((), ())),
        preferred_element_type=jnp.float32)
    ht = jnp.maximum(ht + b1_ref[...], 0.0)
    ot = lax.dot_general(
        w2t_ref[...], ht, (((1,), (0,)), ((), ())),
        preferred_element_type=jnp.float32) + b2_ref[...]
    out_ref[...] = ot[None]


@functools.cache
def _make_mlp_t(B, L, V):
    return pl.pallas_call(
        _mlp_t_body,
        grid=(L,),
        in_specs=[
            pl.BlockSpec((1, 1, B), lambda l: (l, 0, 0)),
            pl.BlockSpec((EMBED_DIM, V), lambda l: (0, 0)),
            pl.BlockSpec((HIDDEN_DIM, EMBED_DIM), lambda l: (0, 0)),
            pl.BlockSpec((HIDDEN_DIM, 1), lambda l: (0, 0)),
            pl.BlockSpec((V, HIDDEN_DIM), lambda l: (0, 0)),
            pl.BlockSpec((V, 1), lambda l: (0, 0)),
        ],
        out_specs=pl.BlockSpec((1, V, B), lambda l: (l, 0, 0)),
        out_shape=jax.ShapeDtypeStruct((L, V, B), jnp.float32),
    )


def kernel(x, emb, W1, b1, W2, b2):
    Bt, L = x.shape
    xt = x.astype(jnp.int32).T.reshape(L, 1, Bt)
    ot = _make_mlp_t(Bt, L, VOCAB)(
        xt, emb.T, W1.T, b1.reshape(HIDDEN_DIM, 1),
        W2.T, b2.reshape(VOCAB, 1))
    return jnp.transpose(ot, (2, 0, 1))


# speedup vs baseline: 8.5498x; 1.1263x over previous
import functools

import jax
import jax.numpy as jnp
from jax import lax
from jax.experimental import pallas as pl

VOCAB = 1000
EMBED_DIM = 64
HIDDEN_DIM = 32


LBLK = 2


def _mlp_t_body(xt_ref, embt_ref, w1t_ref, b1_ref, w2t_ref, b2_ref, out_ref):
    for i in range(LBLK):
        xv = xt_ref[i]
        iota_v = lax.broadcasted_iota(jnp.int32, (VOCAB, xv.shape[1]), 0)
        m = jnp.where(iota_v == xv, 1.0, 0.0)
        et = lax.dot_general(
            embt_ref[...], m, (((1,), (0,)), ((), ())),
            preferred_element_type=jnp.float32)
        ht = lax.dot_general(
            w1t_ref[...], et, (((1,), (0,)), ((), ())),
            preferred_element_type=jnp.float32)
        ht = jnp.maximum(ht + b1_ref[...], 0.0)
        ot = lax.dot_general(
            w2t_ref[...], ht, (((1,), (0,)), ((), ())),
            preferred_element_type=jnp.float32) + b2_ref[...]
        out_ref[i] = ot


@functools.cache
def _make_mlp_t(B, L, V):
    return pl.pallas_call(
        _mlp_t_body,
        grid=(L // LBLK,),
        in_specs=[
            pl.BlockSpec((LBLK, 1, B), lambda l: (l, 0, 0)),
            pl.BlockSpec((EMBED_DIM, V), lambda l: (0, 0)),
            pl.BlockSpec((HIDDEN_DIM, EMBED_DIM), lambda l: (0, 0)),
            pl.BlockSpec((HIDDEN_DIM, 1), lambda l: (0, 0)),
            pl.BlockSpec((V, HIDDEN_DIM), lambda l: (0, 0)),
            pl.BlockSpec((V, 1), lambda l: (0, 0)),
        ],
        out_specs=pl.BlockSpec((LBLK, V, B), lambda l: (l, 0, 0)),
        out_shape=jax.ShapeDtypeStruct((L, V, B), jnp.float32),
    )


def kernel(x, emb, W1, b1, W2, b2):
    Bt, L = x.shape
    xt = x.astype(jnp.int32).T.reshape(L, 1, Bt)
    ot = _make_mlp_t(Bt, L, VOCAB)(
        xt, emb.T, W1.T, b1.reshape(HIDDEN_DIM, 1),
        W2.T, b2.reshape(VOCAB, 1))
    return jnp.transpose(ot, (2, 0, 1))


# speedup vs baseline: 8.5630x; 1.0015x over previous
import functools

import jax
import jax.numpy as jnp
from jax import lax
from jax.experimental import pallas as pl

VOCAB = 1000
EMBED_DIM = 64
HIDDEN_DIM = 32


LBLK = 5


def _mlp_t_body(xt_ref, embt_ref, w1t_ref, b1_ref, w2t_ref, b2_ref, out_ref):
    for i in range(LBLK):
        xv = xt_ref[i]
        iota_v = lax.broadcasted_iota(jnp.int32, (VOCAB, xv.shape[1]), 0)
        m = jnp.where(iota_v == xv, 1.0, 0.0)
        et = lax.dot_general(
            embt_ref[...], m, (((1,), (0,)), ((), ())),
            preferred_element_type=jnp.float32)
        ht = lax.dot_general(
            w1t_ref[...], et, (((1,), (0,)), ((), ())),
            preferred_element_type=jnp.float32)
        ht = jnp.maximum(ht + b1_ref[...], 0.0)
        ot = lax.dot_general(
            w2t_ref[...], ht, (((1,), (0,)), ((), ())),
            preferred_element_type=jnp.float32) + b2_ref[...]
        out_ref[i] = ot


@functools.cache
def _make_mlp_t(B, L, V):
    return pl.pallas_call(
        _mlp_t_body,
        grid=(L // LBLK,),
        in_specs=[
            pl.BlockSpec((LBLK, 1, B), lambda l: (l, 0, 0)),
            pl.BlockSpec((EMBED_DIM, V), lambda l: (0, 0)),
            pl.BlockSpec((HIDDEN_DIM, EMBED_DIM), lambda l: (0, 0)),
            pl.BlockSpec((HIDDEN_DIM, 1), lambda l: (0, 0)),
            pl.BlockSpec((V, HIDDEN_DIM), lambda l: (0, 0)),
            pl.BlockSpec((V, 1), lambda l: (0, 0)),
        ],
        out_specs=pl.BlockSpec((LBLK, V, B), lambda l: (l, 0, 0)),
        out_shape=jax.ShapeDtypeStruct((L, V, B), jnp.float32),
    )


def kernel(x, emb, W1, b1, W2, b2):
    Bt, L = x.shape
    xt = x.astype(jnp.int32).T.reshape(L, 1, Bt)
    ot = _make_mlp_t(Bt, L, VOCAB)(
        xt, emb.T, W1.T, b1.reshape(HIDDEN_DIM, 1),
        W2.T, b2.reshape(VOCAB, 1))
    return jnp.transpose(ot, (2, 0, 1))
